# baseline (speedup 1.0000x reference)
import functools

import jax
import jax.numpy as jnp
from jax import lax
from jax.experimental import pallas as pl
from jax.experimental.pallas import tpu as pltpu
from jax.experimental.pallas import tpu_sc as plsc

NUM_KV = 8
NUM_SC = 2
NUM_TC = NUM_KV - NUM_SC
BEAM = 5
VOCAB = 100000
LANES = 128
VROWS = 782
VPAD = VROWS * LANES
CH = 16
NCHUNK = 16 // CH
NSTEP = NUM_TC * NCHUNK
NBUF = 3


def _sc_rep_body(*refs):
    ins = refs[:NUM_SC]
    outs = refs[NUM_SC:2 * NUM_SC]
    buf = refs[-1]
    c = lax.axis_index("c")
    s = lax.axis_index("s")
    wid = s * 2 + c
    h = wid // 2
    base = (wid % 2) * 32
    for k in range(NUM_SC):
        for q in range(2):
            off = base + q * 16
            pltpu.sync_copy(ins[k].at[h, pl.ds(off, 16)], buf)
            for j in range(BEAM):
                pltpu.sync_copy(buf, outs[k].at[j, h, pl.ds(off, 16)])


def _fused_body(*refs):
    kv_in = refs[:NUM_TC]
    logits_ref, rp_ref, pv_ref = refs[NUM_TC:NUM_TC + 3]
    kv_out = refs[NUM_TC + 3:2 * NUM_TC + 3]
    rp_out_ref, idx_ref, prob_ref = refs[2 * NUM_TC + 3:2 * NUM_TC + 6]
    buf, lsem, ssem = refs[2 * NUM_TC + 6:]

    def load(t):
        k, h = divmod(t, NCHUNK)
        pltpu.make_async_copy(
            kv_in[k].at[0, pl.ds(h * CH, CH)], buf.at[t % NBUF],
            lsem.at[t % NBUF]
        ).start()

    def load_wait(t):
        k, h = divmod(t, NCHUNK)
        pltpu.make_async_copy(
            kv_in[k].at[0, pl.ds(h * CH, CH)], buf.at[t % NBUF],
            lsem.at[t % NBUF]
        ).wait()

    def store(t):
        k, h = divmod(t, NCHUNK)
        for j in range(BEAM):
            pltpu.make_async_copy(
                buf.at[t % NBUF], kv_out[k].at[j, pl.ds(h * CH, CH)],
                ssem.at[t % NBUF]
            ).start()

    def store_wait(t):
        k, h = divmod(t, NCHUNK)
        for j in range(BEAM):
            pltpu.make_async_copy(
                buf.at[t % NBUF], kv_out[k].at[j, pl.ds(h * CH, CH)],
                ssem.at[t % NBUF]
            ).wait()

    for t in range(min(NBUF, NSTEP)):
        load(t)
    load_wait(0)
    store(0)

    x = logits_ref[...]
    rows = jax.lax.broadcasted_iota(jnp.int32, (VROWS, LANES), 0)
    cols = jax.lax.broadcasted_iota(jnp.int32, (VROWS, LANES), 1)
    gidx = rows * LANES + cols
    neg = jnp.float32(-jnp.inf)
    big = jnp.int32(2**30)

    vals = []
    idxs = []
    cur = x
    for _ in range(BEAM):
        m = jnp.max(cur)
        i = jnp.min(jnp.where(cur == m, gidx, big))
        vals.append(m)
        idxs.append(i)
        cur = jnp.where(gidx == i, neg, cur)

    m0 = vals[0]
    s = jnp.sum(jnp.exp(x - m0))
    logz = m0 + jnp.log(s)
    pv = pv_ref[0]

    rp = rp_ref[...]
    vcols = jax.lax.broadcasted_iota(jnp.int32, (BEAM, VOCAB), 1)
    hit = functools.reduce(
        jnp.logical_or, [vcols == idxs[k] for k in range(BEAM)]
    )
    rp_out_ref[...] = jnp.where(hit, rp * pv, rp)

    r8 = jax.lax.broadcasted_iota(jnp.int32, (8, LANES), 0)
    iacc = jnp.zeros((8, LANES), jnp.int32)
    pacc = jnp.zeros((8, LANES), jnp.float32)
    for k in range(BEAM):
        iacc = jnp.where(r8 == k, idxs[k], iacc)
        pacc = jnp.where(r8 == k, vals[k] - logz, pacc)
    idx_ref[...] = iacc
    prob_ref[...] = pacc

    for t in range(NSTEP):
        if t > 0:
            load_wait(t)
            store(t)
        if t + NBUF < NSTEP:
            store_wait(t)
            load(t + NBUF)
    for t in range(max(0, NSTEP - NBUF), NSTEP):
        store_wait(t)


def kernel(kv_0, kv_1, kv_2, kv_3, kv_4, kv_5, kv_6, kv_7,
           save_id, repeat_penality, logits, penality_value, beam_size):
    kvs = [kv_0, kv_1, kv_2, kv_3, kv_4, kv_5, kv_6, kv_7]
    kvs = [jnp.transpose(kv, (0, 1, 3, 2)) for kv in kvs]
    kv_shape = kvs[0].shape

    lpad = jnp.pad(logits, ((0, 0), (0, VPAD - VOCAB)),
                   constant_values=-jnp.inf).reshape(VROWS, LANES)
    pv1 = penality_value.reshape(1)

    sc_mesh = plsc.VectorSubcoreMesh(core_axis_name="c", subcore_axis_name="s")
    sc_in = [kv.reshape(kv_shape[1:]) for kv in kvs[NUM_TC:]]
    sc_saved = functools.partial(
        pl.kernel,
        mesh=sc_mesh,
        out_type=[jax.ShapeDtypeStruct((BEAM,) + kv_shape[1:],
                                       jnp.float32)] * NUM_SC,
        scratch_types=[pltpu.VMEM((16, kv_shape[3]), jnp.float32)],
    )(_sc_rep_body)(*sc_in)

    any_spec = pl.BlockSpec(memory_space=pl.ANY)
    vmem_spec = pl.BlockSpec(memory_space=pltpu.VMEM)
    outs = pl.pallas_call(
        _fused_body,
        in_specs=[any_spec] * NUM_TC + [
            vmem_spec, vmem_spec, pl.BlockSpec(memory_space=pltpu.SMEM)],
        out_specs=[any_spec] * NUM_TC + [vmem_spec, vmem_spec, vmem_spec],
        out_shape=(
            [jax.ShapeDtypeStruct((BEAM,) + kv_shape[1:], jnp.float32)] * NUM_TC
            + [jax.ShapeDtypeStruct((BEAM, VOCAB), jnp.float32),
               jax.ShapeDtypeStruct((8, LANES), jnp.int32),
               jax.ShapeDtypeStruct((8, LANES), jnp.float32)]
        ),
        scratch_shapes=[
            pltpu.VMEM((NBUF, CH) + kv_shape[2:], jnp.float32),
            pltpu.SemaphoreType.DMA((NBUF,)),
            pltpu.SemaphoreType.DMA((NBUF,)),
        ],
    )(*kvs[:NUM_TC], lpad, repeat_penality, pv1)

    saved = [jnp.transpose(o, (0, 1, 3, 2))
             for o in list(outs[:NUM_TC]) + list(sc_saved)]
    rp_out, idx8, prob8 = outs[NUM_TC:]

    tbi = idx8[:BEAM, :1]
    save_id_out = jnp.concatenate([save_id, tbi], axis=-1)
    top_prob = prob8[:BEAM, :1]
    batch_indices = (jnp.arange(BEAM, dtype=jnp.int32)
                     + (jnp.asarray(beam_size, jnp.int32) - jnp.int32(BEAM)))
    max_logits_idx = tbi[0]

    return (*saved, save_id_out, rp_out, top_prob, batch_indices, tbi,
            max_logits_idx)


# speedup vs baseline: 1.0024x; 1.0024x over previous
import functools

import jax
import jax.numpy as jnp
from jax import lax
from jax.experimental import pallas as pl
from jax.experimental.pallas import tpu as pltpu
from jax.experimental.pallas import tpu_sc as plsc

NUM_KV = 8
NUM_SC = 2
NUM_TC = NUM_KV - NUM_SC
BEAM = 5
VOCAB = 100000
LANES = 128
VROWS = 782
VPAD = VROWS * LANES
CH = 16
NCHUNK = 16 // CH
NSTEP = NUM_TC * NCHUNK
NBUF = 3


def _sc_rep_body(*refs):
    ins = refs[:NUM_SC]
    outs = refs[NUM_SC:2 * NUM_SC]
    buf, lsem, ssem = refs[NUM_SC * 2:]
    c = lax.axis_index("c")
    s = lax.axis_index("s")
    wid = s * 2 + c
    h = wid // 2
    base = (wid % 2) * 32
    nch = 2 * NUM_SC

    def src(t):
        k, q = divmod(t, 2)
        return ins[k].at[h, pl.ds(base + q * 16, 16)]

    def load(t):
        pltpu.make_async_copy(src(t), buf.at[t % 2], lsem.at[t % 2]).start()

    def load_wait(t):
        pltpu.make_async_copy(src(t), buf.at[t % 2], lsem.at[t % 2]).wait()

    def store(t):
        k, q = divmod(t, 2)
        for j in range(BEAM):
            pltpu.make_async_copy(
                buf.at[t % 2], outs[k].at[j, h, pl.ds(base + q * 16, 16)],
                ssem.at[t % 2]).start()

    def store_wait(t):
        k, q = divmod(t, 2)
        for j in range(BEAM):
            pltpu.make_async_copy(
                buf.at[t % 2], outs[k].at[j, h, pl.ds(base + q * 16, 16)],
                ssem.at[t % 2]).wait()

    load(0)
    load(1)
    for t in range(nch):
        load_wait(t)
        store(t)
        if t + 2 < nch:
            store_wait(t)
            load(t + 2)
    store_wait(nch - 2)
    store_wait(nch - 1)


def _fused_body(*refs):
    kv_in = refs[:NUM_TC]
    logits_ref, rp_ref, pv_ref = refs[NUM_TC:NUM_TC + 3]
    kv_out = refs[NUM_TC + 3:2 * NUM_TC + 3]
    rp_out_ref, idx_ref, prob_ref = refs[2 * NUM_TC + 3:2 * NUM_TC + 6]
    buf, lsem, ssem = refs[2 * NUM_TC + 6:]

    def load(t):
        k, h = divmod(t, NCHUNK)
        pltpu.make_async_copy(
            kv_in[k].at[0, pl.ds(h * CH, CH)], buf.at[t % NBUF],
            lsem.at[t % NBUF]
        ).start()

    def load_wait(t):
        k, h = divmod(t, NCHUNK)
        pltpu.make_async_copy(
            kv_in[k].at[0, pl.ds(h * CH, CH)], buf.at[t % NBUF],
            lsem.at[t % NBUF]
        ).wait()

    def store(t):
        k, h = divmod(t, NCHUNK)
        for j in range(BEAM):
            pltpu.make_async_copy(
                buf.at[t % NBUF], kv_out[k].at[j, pl.ds(h * CH, CH)],
                ssem.at[t % NBUF]
            ).start()

    def store_wait(t):
        k, h = divmod(t, NCHUNK)
        for j in range(BEAM):
            pltpu.make_async_copy(
                buf.at[t % NBUF], kv_out[k].at[j, pl.ds(h * CH, CH)],
                ssem.at[t % NBUF]
            ).wait()

    for t in range(min(NBUF, NSTEP)):
        load(t)
    load_wait(0)
    store(0)

    x = logits_ref[...]
    rows = jax.lax.broadcasted_iota(jnp.int32, (VROWS, LANES), 0)
    cols = jax.lax.broadcasted_iota(jnp.int32, (VROWS, LANES), 1)
    gidx = rows * LANES + cols
    neg = jnp.float32(-jnp.inf)
    big = jnp.int32(2**30)

    vals = []
    idxs = []
    cur = x
    for _ in range(BEAM):
        m = jnp.max(cur)
        i = jnp.min(jnp.where(cur == m, gidx, big))
        vals.append(m)
        idxs.append(i)
        cur = jnp.where(gidx == i, neg, cur)

    m0 = vals[0]
    s = jnp.sum(jnp.exp(x - m0))
    logz = m0 + jnp.log(s)
    pv = pv_ref[0]

    rp = rp_ref[...]
    vcols = jax.lax.broadcasted_iota(jnp.int32, (BEAM, VOCAB), 1)
    hit = functools.reduce(
        jnp.logical_or, [vcols == idxs[k] for k in range(BEAM)]
    )
    rp_out_ref[...] = jnp.where(hit, rp * pv, rp)

    r8 = jax.lax.broadcasted_iota(jnp.int32, (8, LANES), 0)
    iacc = jnp.zeros((8, LANES), jnp.int32)
    pacc = jnp.zeros((8, LANES), jnp.float32)
    for k in range(BEAM):
        iacc = jnp.where(r8 == k, idxs[k], iacc)
        pacc = jnp.where(r8 == k, vals[k] - logz, pacc)
    idx_ref[...] = iacc
    prob_ref[...] = pacc

    for t in range(NSTEP):
        if t > 0:
            load_wait(t)
            store(t)
        if t + NBUF < NSTEP:
            store_wait(t)
            load(t + NBUF)
    for t in range(max(0, NSTEP - NBUF), NSTEP):
        store_wait(t)


def kernel(kv_0, kv_1, kv_2, kv_3, kv_4, kv_5, kv_6, kv_7,
           save_id, repeat_penality, logits, penality_value, beam_size):
    kvs = [kv_0, kv_1, kv_2, kv_3, kv_4, kv_5, kv_6, kv_7]
    kvs = [jnp.transpose(kv, (0, 1, 3, 2)) for kv in kvs]
    kv_shape = kvs[0].shape

    lpad = jnp.pad(logits, ((0, 0), (0, VPAD - VOCAB)),
                   constant_values=-jnp.inf).reshape(VROWS, LANES)
    pv1 = penality_value.reshape(1)

    sc_mesh = plsc.VectorSubcoreMesh(core_axis_name="c", subcore_axis_name="s")
    sc_in = [kv.reshape(kv_shape[1:]) for kv in kvs[NUM_TC:]]
    sc_saved = functools.partial(
        pl.kernel,
        mesh=sc_mesh,
        out_type=[jax.ShapeDtypeStruct((BEAM,) + kv_shape[1:],
                                       jnp.float32)] * NUM_SC,
        scratch_types=[pltpu.VMEM((2, 16, kv_shape[3]), jnp.float32),
                       pltpu.SemaphoreType.DMA((2,)),
                       pltpu.SemaphoreType.DMA((2,))],
    )(_sc_rep_body)(*sc_in)

    any_spec = pl.BlockSpec(memory_space=pl.ANY)
    vmem_spec = pl.BlockSpec(memory_space=pltpu.VMEM)
    outs = pl.pallas_call(
        _fused_body,
        in_specs=[any_spec] * NUM_TC + [
            vmem_spec, vmem_spec, pl.BlockSpec(memory_space=pltpu.SMEM)],
        out_specs=[any_spec] * NUM_TC + [vmem_spec, vmem_spec, vmem_spec],
        out_shape=(
            [jax.ShapeDtypeStruct((BEAM,) + kv_shape[1:], jnp.float32)] * NUM_TC
            + [jax.ShapeDtypeStruct((BEAM, VOCAB), jnp.float32),
               jax.ShapeDtypeStruct((8, LANES), jnp.int32),
               jax.ShapeDtypeStruct((8, LANES), jnp.float32)]
        ),
        scratch_shapes=[
            pltpu.VMEM((NBUF, CH) + kv_shape[2:], jnp.float32),
            pltpu.SemaphoreType.DMA((NBUF,)),
            pltpu.SemaphoreType.DMA((NBUF,)),
        ],
    )(*kvs[:NUM_TC], lpad, repeat_penality, pv1)

    saved = [jnp.transpose(o, (0, 1, 3, 2))
             for o in list(outs[:NUM_TC]) + list(sc_saved)]
    rp_out, idx8, prob8 = outs[NUM_TC:]

    tbi = idx8[:BEAM, :1]
    save_id_out = jnp.concatenate([save_id, tbi], axis=-1)
    top_prob = prob8[:BEAM, :1]
    batch_indices = (jnp.arange(BEAM, dtype=jnp.int32)
                     + (jnp.asarray(beam_size, jnp.int32) - jnp.int32(BEAM)))
    max_logits_idx = tbi[0]

    return (*saved, save_id_out, rp_out, top_prob, batch_indices, tbi,
            max_logits_idx)


# speedup vs baseline: 1.0140x; 1.0116x over previous
import functools

import jax
import jax.numpy as jnp
from jax import lax
from jax.experimental import pallas as pl
from jax.experimental.pallas import tpu as pltpu
from jax.experimental.pallas import tpu_sc as plsc

NUM_KV = 8
NUM_SC = 1
NUM_TC = NUM_KV - NUM_SC
BEAM = 5
VOCAB = 100000
LANES = 128
VROWS = 782
VPAD = VROWS * LANES
CH = 16
NCHUNK = 16 // CH
NSTEP = NUM_TC * NCHUNK
NBUF = 3


def _sc_rep_body(*refs):
    ins = refs[:NUM_SC]
    outs = refs[NUM_SC:2 * NUM_SC]
    buf, lsem, ssem = refs[NUM_SC * 2:]
    c = lax.axis_index("c")
    s = lax.axis_index("s")
    wid = s * 2 + c
    h = wid // 2
    base = (wid % 2) * 32
    nch = 2 * NUM_SC

    def src(t):
        k, q = divmod(t, 2)
        return ins[k].at[h, pl.ds(base + q * 16, 16)]

    def load(t):
        pltpu.make_async_copy(src(t), buf.at[t % 2], lsem.at[t % 2]).start()

    def load_wait(t):
        pltpu.make_async_copy(src(t), buf.at[t % 2], lsem.at[t % 2]).wait()

    def store(t):
        k, q = divmod(t, 2)
        for j in range(BEAM):
            pltpu.make_async_copy(
                buf.at[t % 2], outs[k].at[j, h, pl.ds(base + q * 16, 16)],
                ssem.at[t % 2]).start()

    def store_wait(t):
        k, q = divmod(t, 2)
        for j in range(BEAM):
            pltpu.make_async_copy(
                buf.at[t % 2], outs[k].at[j, h, pl.ds(base + q * 16, 16)],
                ssem.at[t % 2]).wait()

    load(0)
    load(1)
    for t in range(nch):
        load_wait(t)
        store(t)
        if t + 2 < nch:
            store_wait(t)
            load(t + 2)
    store_wait(nch - 2)
    store_wait(nch - 1)


def _fused_body(*refs):
    kv_in = refs[:NUM_TC]
    logits_ref, rp_ref, pv_ref = refs[NUM_TC:NUM_TC + 3]
    kv_out = refs[NUM_TC + 3:2 * NUM_TC + 3]
    rp_out_ref, idx_ref, prob_ref = refs[2 * NUM_TC + 3:2 * NUM_TC + 6]
    buf, lsem, ssem = refs[2 * NUM_TC + 6:]

    def load(t):
        k, h = divmod(t, NCHUNK)
        pltpu.make_async_copy(
            kv_in[k].at[0, pl.ds(h * CH, CH)], buf.at[t % NBUF],
            lsem.at[t % NBUF]
        ).start()

    def load_wait(t):
        k, h = divmod(t, NCHUNK)
        pltpu.make_async_copy(
            kv_in[k].at[0, pl.ds(h * CH, CH)], buf.at[t % NBUF],
            lsem.at[t % NBUF]
        ).wait()

    def store(t):
        k, h = divmod(t, NCHUNK)
        for j in range(BEAM):
            pltpu.make_async_copy(
                buf.at[t % NBUF], kv_out[k].at[j, pl.ds(h * CH, CH)],
                ssem.at[t % NBUF]
            ).start()

    def store_wait(t):
        k, h = divmod(t, NCHUNK)
        for j in range(BEAM):
            pltpu.make_async_copy(
                buf.at[t % NBUF], kv_out[k].at[j, pl.ds(h * CH, CH)],
                ssem.at[t % NBUF]
            ).wait()

    for t in range(min(NBUF, NSTEP)):
        load(t)
    load_wait(0)
    store(0)

    x = logits_ref[...]
    rows = jax.lax.broadcasted_iota(jnp.int32, (VROWS, LANES), 0)
    cols = jax.lax.broadcasted_iota(jnp.int32, (VROWS, LANES), 1)
    gidx = rows * LANES + cols
    neg = jnp.float32(-jnp.inf)
    big = jnp.int32(2**30)

    vals = []
    idxs = []
    cur = x
    for _ in range(BEAM):
        m = jnp.max(cur)
        i = jnp.min(jnp.where(cur == m, gidx, big))
        vals.append(m)
        idxs.append(i)
        cur = jnp.where(gidx == i, neg, cur)

    m0 = vals[0]
    s = jnp.sum(jnp.exp(x - m0))
    logz = m0 + jnp.log(s)
    pv = pv_ref[0]

    rp = rp_ref[...]
    vcols = jax.lax.broadcasted_iota(jnp.int32, (BEAM, VOCAB), 1)
    hit = functools.reduce(
        jnp.logical_or, [vcols == idxs[k] for k in range(BEAM)]
    )
    rp_out_ref[...] = jnp.where(hit, rp * pv, rp)

    r8 = jax.lax.broadcasted_iota(jnp.int32, (8, LANES), 0)
    iacc = jnp.zeros((8, LANES), jnp.int32)
    pacc = jnp.zeros((8, LANES), jnp.float32)
    for k in range(BEAM):
        iacc = jnp.where(r8 == k, idxs[k], iacc)
        pacc = jnp.where(r8 == k, vals[k] - logz, pacc)
    idx_ref[...] = iacc
    prob_ref[...] = pacc

    for t in range(NSTEP):
        if t > 0:
            load_wait(t)
            store(t)
        if t + NBUF < NSTEP:
            store_wait(t)
            load(t + NBUF)
    for t in range(max(0, NSTEP - NBUF), NSTEP):
        store_wait(t)


def kernel(kv_0, kv_1, kv_2, kv_3, kv_4, kv_5, kv_6, kv_7,
           save_id, repeat_penality, logits, penality_value, beam_size):
    kvs = [kv_0, kv_1, kv_2, kv_3, kv_4, kv_5, kv_6, kv_7]
    kvs = [jnp.transpose(kv, (0, 1, 3, 2)) for kv in kvs]
    kv_shape = kvs[0].shape

    lpad = jnp.pad(logits, ((0, 0), (0, VPAD - VOCAB)),
                   constant_values=-jnp.inf).reshape(VROWS, LANES)
    pv1 = penality_value.reshape(1)

    sc_mesh = plsc.VectorSubcoreMesh(core_axis_name="c", subcore_axis_name="s")
    sc_in = [kv.reshape(kv_shape[1:]) for kv in kvs[NUM_TC:]]
    sc_saved = functools.partial(
        pl.kernel,
        mesh=sc_mesh,
        out_type=[jax.ShapeDtypeStruct((BEAM,) + kv_shape[1:],
                                       jnp.float32)] * NUM_SC,
        scratch_types=[pltpu.VMEM((2, 16, kv_shape[3]), jnp.float32),
                       pltpu.SemaphoreType.DMA((2,)),
                       pltpu.SemaphoreType.DMA((2,))],
    )(_sc_rep_body)(*sc_in)

    any_spec = pl.BlockSpec(memory_space=pl.ANY)
    vmem_spec = pl.BlockSpec(memory_space=pltpu.VMEM)
    outs = pl.pallas_call(
        _fused_body,
        in_specs=[any_spec] * NUM_TC + [
            vmem_spec, vmem_spec, pl.BlockSpec(memory_space=pltpu.SMEM)],
        out_specs=[any_spec] * NUM_TC + [vmem_spec, vmem_spec, vmem_spec],
        out_shape=(
            [jax.ShapeDtypeStruct((BEAM,) + kv_shape[1:], jnp.float32)] * NUM_TC
            + [jax.ShapeDtypeStruct((BEAM, VOCAB), jnp.float32),
               jax.ShapeDtypeStruct((8, LANES), jnp.int32),
               jax.ShapeDtypeStruct((8, LANES), jnp.float32)]
        ),
        scratch_shapes=[
            pltpu.VMEM((NBUF, CH) + kv_shape[2:], jnp.float32),
            pltpu.SemaphoreType.DMA((NBUF,)),
            pltpu.SemaphoreType.DMA((NBUF,)),
        ],
    )(*kvs[:NUM_TC], lpad, repeat_penality, pv1)

    saved = [jnp.transpose(o, (0, 1, 3, 2))
             for o in list(outs[:NUM_TC]) + list(sc_saved)]
    rp_out, idx8, prob8 = outs[NUM_TC:]

    tbi = idx8[:BEAM, :1]
    save_id_out = jnp.concatenate([save_id, tbi], axis=-1)
    top_prob = prob8[:BEAM, :1]
    batch_indices = (jnp.arange(BEAM, dtype=jnp.int32)
                     + (jnp.asarray(beam_size, jnp.int32) - jnp.int32(BEAM)))
    max_logits_idx = tbi[0]

    return (*saved, save_id_out, rp_out, top_prob, batch_indices, tbi,
            max_logits_idx)


# speedup vs baseline: 1.1299x; 1.1143x over previous
import functools

import jax
import jax.numpy as jnp
from jax.experimental import pallas as pl
from jax.experimental.pallas import tpu as pltpu

NUM_KV = 8
BEAM = 5
VOCAB = 100000
LANES = 128
VROWS = 782
VPAD = VROWS * LANES
CH = 16
NCHUNK = 16 // CH
NSTEP = NUM_KV * NCHUNK
NBUF = 3


def _fused_body(*refs):
    kv_in = refs[:NUM_KV]
    logits_ref, rp_ref, pv_ref = refs[NUM_KV:NUM_KV + 3]
    kv_out = refs[NUM_KV + 3:2 * NUM_KV + 3]
    rp_out_ref, idx_ref, prob_ref = refs[2 * NUM_KV + 3:2 * NUM_KV + 6]
    buf, lsem, ssem = refs[2 * NUM_KV + 6:]

    def load(t):
        k, h = divmod(t, NCHUNK)
        pltpu.make_async_copy(
            kv_in[k].at[0, pl.ds(h * CH, CH)], buf.at[t % NBUF],
            lsem.at[t % NBUF]
        ).start()

    def load_wait(t):
        k, h = divmod(t, NCHUNK)
        pltpu.make_async_copy(
            kv_in[k].at[0, pl.ds(h * CH, CH)], buf.at[t % NBUF],
            lsem.at[t % NBUF]
        ).wait()

    def store(t):
        k, h = divmod(t, NCHUNK)
        for j in range(BEAM):
            pltpu.make_async_copy(
                buf.at[t % NBUF], kv_out[k].at[j, pl.ds(h * CH, CH)],
                ssem.at[t % NBUF]
            ).start()

    def store_wait(t):
        k, h = divmod(t, NCHUNK)
        for j in range(BEAM):
            pltpu.make_async_copy(
                buf.at[t % NBUF], kv_out[k].at[j, pl.ds(h * CH, CH)],
                ssem.at[t % NBUF]
            ).wait()

    for t in range(min(NBUF, NSTEP)):
        load(t)
    load_wait(0)
    store(0)

    x = logits_ref[...]
    rows = jax.lax.broadcasted_iota(jnp.int32, (VROWS, LANES), 0)
    cols = jax.lax.broadcasted_iota(jnp.int32, (VROWS, LANES), 1)
    gidx = rows * LANES + cols
    neg = jnp.float32(-jnp.inf)
    big = jnp.int32(2**30)

    vals = []
    idxs = []
    cur = x
    for _ in range(BEAM):
        m = jnp.max(cur)
        i = jnp.min(jnp.where(cur == m, gidx, big))
        vals.append(m)
        idxs.append(i)
        cur = jnp.where(gidx == i, neg, cur)

    m0 = vals[0]
    s = jnp.sum(jnp.exp(x - m0))
    logz = m0 + jnp.log(s)
    pv = pv_ref[0]

    rp = rp_ref[...]
    vcols = jax.lax.broadcasted_iota(jnp.int32, (BEAM, VOCAB), 1)
    hit = functools.reduce(
        jnp.logical_or, [vcols == idxs[k] for k in range(BEAM)]
    )
    rp_out_ref[...] = jnp.where(hit, rp * pv, rp)

    r8 = jax.lax.broadcasted_iota(jnp.int32, (8, LANES), 0)
    iacc = jnp.zeros((8, LANES), jnp.int32)
    pacc = jnp.zeros((8, LANES), jnp.float32)
    for k in range(BEAM):
        iacc = jnp.where(r8 == k, idxs[k], iacc)
        pacc = jnp.where(r8 == k, vals[k] - logz, pacc)
    idx_ref[...] = iacc
    prob_ref[...] = pacc

    for t in range(NSTEP):
        if t > 0:
            load_wait(t)
            store(t)
        if t + NBUF < NSTEP:
            store_wait(t)
            load(t + NBUF)
    for t in range(max(0, NSTEP - NBUF), NSTEP):
        store_wait(t)


def kernel(kv_0, kv_1, kv_2, kv_3, kv_4, kv_5, kv_6, kv_7,
           save_id, repeat_penality, logits, penality_value, beam_size):
    kvs = [kv_0, kv_1, kv_2, kv_3, kv_4, kv_5, kv_6, kv_7]
    kvs = [jnp.transpose(kv, (0, 1, 3, 2)) for kv in kvs]
    kv_shape = kvs[0].shape

    lpad = jnp.pad(logits, ((0, 0), (0, VPAD - VOCAB)),
                   constant_values=-jnp.inf).reshape(VROWS, LANES)
    pv1 = penality_value.reshape(1)

    any_spec = pl.BlockSpec(memory_space=pl.ANY)
    vmem_spec = pl.BlockSpec(memory_space=pltpu.VMEM)
    outs = pl.pallas_call(
        _fused_body,
        in_specs=[any_spec] * NUM_KV + [
            vmem_spec, vmem_spec, pl.BlockSpec(memory_space=pltpu.SMEM)],
        out_specs=[any_spec] * NUM_KV + [vmem_spec, vmem_spec, vmem_spec],
        out_shape=(
            [jax.ShapeDtypeStruct((BEAM,) + kv_shape[1:], jnp.float32)] * NUM_KV
            + [jax.ShapeDtypeStruct((BEAM, VOCAB), jnp.float32),
               jax.ShapeDtypeStruct((8, LANES), jnp.int32),
               jax.ShapeDtypeStruct((8, LANES), jnp.float32)]
        ),
        scratch_shapes=[
            pltpu.VMEM((NBUF, CH) + kv_shape[2:], jnp.float32),
            pltpu.SemaphoreType.DMA((NBUF,)),
            pltpu.SemaphoreType.DMA((NBUF,)),
        ],
    )(*kvs, lpad, repeat_penality, pv1)

    saved = [jnp.transpose(o, (0, 1, 3, 2)) for o in outs[:NUM_KV]]
    rp_out, idx8, prob8 = outs[NUM_KV:]

    tbi = idx8[:BEAM, :1]
    save_id_out = jnp.concatenate([save_id, tbi], axis=-1)
    top_prob = prob8[:BEAM, :1]
    batch_indices = (jnp.arange(BEAM, dtype=jnp.int32)
                     + (jnp.asarray(beam_size, jnp.int32) - jnp.int32(BEAM)))
    max_logits_idx = tbi[0]

    return (*saved, save_id_out, rp_out, top_prob, batch_indices, tbi,
            max_logits_idx)
